# baseline (speedup 1.0000x reference)
import functools

import jax
import jax.numpy as jnp
from jax import lax
from jax.experimental import pallas as pl
from jax.experimental.pallas import tpu as pltpu
from jax.experimental.pallas import tpu_sc as plsc

D_MODEL = 768
N_TOKENS = 4 * 8192
LANES = 16
CHUNK = 64
EPS = 1e-5


_GATHER_DNUMS = lax.GatherDimensionNumbers(
    offset_dims=(), collapsed_slice_dims=(0,), start_index_map=(0,))


def _shuffle(v, idx):
    return lax.gather(v, idx[:, None], _GATHER_DNUMS, slice_sizes=(1,),
                      mode=lax.GatherScatterMode.PROMISE_IN_BOUNDS)


def _lane_sum_all(v):
    idx = lax.iota(jnp.int32, LANES)
    for k in (8, 4, 2, 1):
        v = v + _shuffle(v, jnp.bitwise_xor(idx, k))
    return v


def _rsqrt_scalar(x):
    i = lax.bitcast_convert_type(x, jnp.int32)
    i = jnp.int32(0x5F3759DF) - lax.shift_right_logical(i, 1)
    y = lax.bitcast_convert_type(i, jnp.float32)
    half_x = x * jnp.float32(0.5)
    for _ in range(3):
        y = y * (jnp.float32(1.5) - half_x * y * y)
    return y


def _make_sc_kernel(n_workers):
    rows_per_w = N_TOKENS // n_workers
    n_chunks = rows_per_w // CHUNK
    nvec = D_MODEL // LANES
    mesh = plsc.VectorSubcoreMesh(core_axis_name="c", subcore_axis_name="s")

    @functools.partial(
        pl.kernel,
        mesh=mesh,
        out_type=jax.ShapeDtypeStruct((N_TOKENS, D_MODEL), jnp.float32),
        scratch_types=[
            pltpu.VMEM((CHUNK,), jnp.int32),
            pltpu.VMEM((CHUNK, D_MODEL), jnp.float32),
            pltpu.VMEM((D_MODEL,), jnp.float32),
            pltpu.VMEM((D_MODEL,), jnp.float32),
            pltpu.SemaphoreType.DMA,
        ],
    )
    def embed_ln(idx_hbm, table_hbm, gamma_hbm, beta_hbm, out_hbm,
                 idx_v, rows_v, gamma_v, beta_v, sem):
        n_cores = 2
        wid = lax.axis_index("s") * n_cores + lax.axis_index("c")
        pltpu.sync_copy(gamma_hbm, gamma_v)
        pltpu.sync_copy(beta_hbm, beta_v)

        def chunk_body(c, _):
            base = wid * rows_per_w + c * CHUNK
            pltpu.sync_copy(idx_hbm.at[pl.ds(base, CHUNK)], idx_v)
            pltpu.async_copy(table_hbm.at[idx_v], rows_v, sem).wait()

            def row_body(r, _):
                def stat_body(j, carry):
                    acc, acc2 = carry
                    v = rows_v[r, pl.ds(j * LANES, LANES)]
                    return acc + v, acc2 + v * v

                zero = jnp.zeros((LANES,), jnp.float32)
                acc, acc2 = lax.fori_loop(0, nvec, stat_body, (zero, zero))
                mean_v = _lane_sum_all(acc) * (1.0 / D_MODEL)
                var_v = _lane_sum_all(acc2) * (1.0 / D_MODEL) - mean_v * mean_v
                var_s = (var_v + EPS)[0]
                rstd_v = jnp.full((LANES,), _rsqrt_scalar(var_s), jnp.float32)

                def norm_body(j, _):
                    sl = pl.ds(j * LANES, LANES)
                    v = rows_v[r, sl]
                    rows_v[r, sl] = (v - mean_v) * rstd_v * gamma_v[sl] + beta_v[sl]
                    return 0

                lax.fori_loop(0, nvec, norm_body, 0)
                return 0

            lax.fori_loop(0, CHUNK, row_body, 0)
            pltpu.sync_copy(rows_v, out_hbm.at[pl.ds(base, CHUNK)])
            return 0

        lax.fori_loop(0, n_chunks, chunk_body, 0)

    return embed_ln


def kernel(tokens, W_E, ln_gamma, ln_beta):
    info = plsc.get_sparse_core_info()
    n_workers = info.num_cores * info.num_subcores
    idx = tokens.reshape(-1).astype(jnp.int32)
    out = _make_sc_kernel(n_workers)(idx, W_E, ln_gamma, ln_beta)
    return out.reshape(tokens.shape[0], tokens.shape[1], D_MODEL)
